# baseline (speedup 1.0000x reference)
import functools

import jax
import jax.numpy as jnp
from jax import lax
from jax.experimental import pallas as pl
from jax.experimental.pallas import tpu as pltpu
from jax.experimental.pallas import tpu_sc as plsc

EPS_BN = 1e-5
SLOPE = 0.01

N = 10000
E = 320000
NC = 2
NS = 16
NW = NC * NS
EPT = E // NW
K = 125
NCHUNK = EPT // K
NPAD = 10240


def _leaky(v):
    return jnp.where(v >= 0, v, SLOPE * v)


def _mm_act_kernel(a_ref, w_ref, b_ref, o_ref, *, act):
    y = jnp.dot(a_ref[...], w_ref[...], preferred_element_type=jnp.float32)
    y = y + b_ref[...][None, :]
    if act == "leaky":
        y = _leaky(y)
    elif act == "relu":
        y = jnp.maximum(y, 0.0)
    o_ref[...] = y


def _mm_act(a, w, b, act, block_rows):
    m, k = a.shape
    n = w.shape[1]
    grid = m // block_rows
    return pl.pallas_call(
        functools.partial(_mm_act_kernel, act=act),
        grid=(grid,),
        in_specs=[
            pl.BlockSpec((block_rows, k), lambda i: (i, 0)),
            pl.BlockSpec((k, n), lambda i: (0, 0)),
            pl.BlockSpec((n,), lambda i: (0,)),
        ],
        out_specs=pl.BlockSpec((block_rows, n), lambda i: (i, 0)),
        out_shape=jax.ShapeDtypeStruct((m, n), jnp.float32),
    )(a, w, b)


def _fc_gcn_kernel(h_ref, wfc_ref, bfc_ref, wg_ref, deg_ref, o_ref):
    h0 = jnp.dot(h_ref[...], wfc_ref[...], preferred_element_type=jnp.float32)
    h0 = jnp.maximum(h0 + bfc_ref[...][None, :], 0.0)
    deg = jnp.sum(deg_ref[...][0], axis=0) + 1.0
    dinv = lax.rsqrt(deg)
    xs = dinv[:, None] * jnp.dot(h0, wg_ref[...],
                                 preferred_element_type=jnp.float32)
    o_ref[...] = xs


def _fc_gcn(h2, wfc, bfc, wg, degs, block_rows=400):
    grid = N // block_rows
    return pl.pallas_call(
        _fc_gcn_kernel,
        grid=(grid,),
        in_specs=[
            pl.BlockSpec((block_rows, 288), lambda i: (i, 0)),
            pl.BlockSpec((288, 128), lambda i: (0, 0)),
            pl.BlockSpec((128,), lambda i: (0,)),
            pl.BlockSpec((128, 128), lambda i: (0, 0)),
            pl.BlockSpec((1, NW, block_rows), lambda i: (i, 0, 0)),
        ],
        out_specs=pl.BlockSpec((block_rows, 128), lambda i: (i, 0)),
        out_shape=jax.ShapeDtypeStruct((N, 128), jnp.float32),
    )(h2, wfc, bfc, wg, degs)


def _combine_kernel(acc_ref, xs_ref, deg_ref, bg_ref, wg_ref, o_ref):
    deg = jnp.sum(deg_ref[...][0], axis=0) + 1.0
    dinv = lax.rsqrt(deg)
    tot = acc_ref[0] + acc_ref[1] + xs_ref[...]
    h = jnp.maximum(dinv[:, None] * tot + bg_ref[...][None, :], 0.0)
    xs2 = dinv[:, None] * jnp.dot(h, wg_ref[...],
                                  preferred_element_type=jnp.float32)
    o_ref[...] = xs2


def _combine_next(acc, xs, degs, bg, wg, block_rows=400):
    grid = N // block_rows
    return pl.pallas_call(
        _combine_kernel,
        grid=(grid,),
        in_specs=[
            pl.BlockSpec((2, block_rows, 128), lambda i: (0, i, 0)),
            pl.BlockSpec((block_rows, 128), lambda i: (i, 0)),
            pl.BlockSpec((1, NW, block_rows), lambda i: (i, 0, 0)),
            pl.BlockSpec((128,), lambda i: (0,)),
            pl.BlockSpec((128, 128), lambda i: (0, 0)),
        ],
        out_specs=pl.BlockSpec((block_rows, 128), lambda i: (i, 0)),
        out_shape=jax.ShapeDtypeStruct((N, 128), jnp.float32),
    )(acc, xs, degs, bg, wg)


def _head_kernel(acc_ref, xs_ref, deg_ref, bg_ref, wo_ref, bo_ref, o_ref):
    deg = jnp.sum(deg_ref[...][0], axis=0) + 1.0
    dinv = lax.rsqrt(deg)
    tot = acc_ref[0] + acc_ref[1] + xs_ref[...]
    h = jnp.maximum(dinv[:, None] * tot + bg_ref[...][None, :], 0.0)
    z = jnp.dot(h, wo_ref[...], preferred_element_type=jnp.float32)
    z = z + bo_ref[...][None, :]
    z = z - jnp.max(z, axis=1, keepdims=True)
    ez = jnp.exp(z)
    o_ref[...] = ez / jnp.sum(ez, axis=1, keepdims=True)


def _head(acc, xs, degs, bg, wo, bo, block_rows=400):
    grid = N // block_rows
    return pl.pallas_call(
        _head_kernel,
        grid=(grid,),
        in_specs=[
            pl.BlockSpec((2, block_rows, 128), lambda i: (0, i, 0)),
            pl.BlockSpec((block_rows, 128), lambda i: (i, 0)),
            pl.BlockSpec((1, NW, block_rows), lambda i: (i, 0, 0)),
            pl.BlockSpec((128,), lambda i: (0,)),
            pl.BlockSpec((128, 64), lambda i: (0, 0)),
            pl.BlockSpec((64,), lambda i: (0,)),
        ],
        out_specs=pl.BlockSpec((block_rows, 64), lambda i: (i, 0)),
        out_shape=jax.ShapeDtypeStruct((N, 64), jnp.float32),
    )(acc, xs, degs, bg, wo, bo)


def _sc_mesh():
    return plsc.VectorSubcoreMesh(core_axis_name="c", subcore_axis_name="s",
                                  num_cores=NC, num_subcores=NS)


def _degree(col16):
    rows_per_tile = EPT // 16

    @functools.partial(
        pl.kernel,
        out_type=jax.ShapeDtypeStruct((NW, N), jnp.float32),
        mesh=_sc_mesh(),
        compiler_params=pltpu.CompilerParams(needs_layout_passes=False),
        scratch_types=[
            pltpu.VMEM((rows_per_tile, 16), jnp.int32),
            pltpu.VMEM((N,), jnp.float32),
        ],
    )
    def run(col_hbm, out_hbm, col_v, deg_v):
        c = lax.axis_index("c")
        s = lax.axis_index("s")
        wid = c * NS + s
        pltpu.sync_copy(col_hbm.at[wid], col_v)
        zeros = jnp.zeros((16,), jnp.float32)

        def zbody(i, _):
            deg_v[pl.ds(i * 16, 16)] = zeros
            return 0

        lax.fori_loop(0, N // 16, zbody, 0)
        ones = jnp.ones((16,), jnp.float32)

        def body(i, _):
            idx = col_v[i, :]
            plsc.addupdate_scatter(deg_v, [idx], ones)
            return 0

        lax.fori_loop(0, rows_per_tile, body, 0)
        pltpu.sync_copy(deg_v, out_hbm.at[wid])

    return run(col16)


def _propagate(xs, row2d, col2d, zrows):
    nslice = NPAD // NS
    G = 16
    NGROUP = NCHUNK // G

    @functools.partial(
        pl.kernel,
        out_type=jax.ShapeDtypeStruct((NC, NPAD, 128), jnp.float32),
        mesh=_sc_mesh(),
        compiler_params=pltpu.CompilerParams(needs_layout_passes=False),
        scratch_types=[
            pltpu.VMEM((G, K), jnp.int32),
            pltpu.VMEM((G, K), jnp.int32),
            pltpu.VMEM((K, 128), jnp.float32),
            pltpu.VMEM((K, 128), jnp.float32),
            pltpu.VMEM_SHARED((NPAD, 128), jnp.float32),
            pltpu.SemaphoreType.DMA,
            pltpu.SemaphoreType.DMA,
        ],
    )
    def run(xs_hbm, row_hbm, col_hbm, z_hbm, out_hbm,
            row_v, col_v, bufa, bufb, acc, sema, semb):
        c = lax.axis_index("c")
        s = lax.axis_index("s")
        wid = c * NS + s
        pltpu.sync_copy(z_hbm, acc.at[pl.ds(s * nslice, nslice)])
        plsc.subcore_barrier()

        def wait(buf, sem):
            pltpu.make_async_copy(xs_hbm.at[row_v.at[0]], buf, sem).wait()

        def group(g, _):
            base = wid * NCHUNK + g * G
            pltpu.sync_copy(row_hbm.at[pl.ds(base, G)], row_v)
            pltpu.sync_copy(col_hbm.at[pl.ds(base, G)], col_v)
            pltpu.async_copy(xs_hbm.at[row_v.at[0]], bufa, sema)

            def body(t, _):
                j = t * 2
                pltpu.async_copy(xs_hbm.at[row_v.at[j + 1]], bufb, semb)
                wait(bufa, sema)
                pltpu.sync_copy(bufa, acc.at[col_v.at[j]], add=True)

                @pl.when(t + 1 < G // 2)
                def _():
                    pltpu.async_copy(xs_hbm.at[row_v.at[j + 2]], bufa, sema)

                wait(bufb, semb)
                pltpu.sync_copy(bufb, acc.at[col_v.at[j + 1]], add=True)
                return 0

            lax.fori_loop(0, G // 2, body, 0)
            return 0

        lax.fori_loop(0, NGROUP, group, 0)
        plsc.subcore_barrier()
        pltpu.sync_copy(acc.at[pl.ds(s * nslice, nslice)],
                        out_hbm.at[c].at[pl.ds(s * nslice, nslice)])

    return run(xs, row2d, col2d, zrows)


def kernel(x, edge_index, W_conv1, b_conv1, gamma1, beta1,
           W_conv2, b_conv2, gamma2, beta2,
           W_fc, b_fc, W_g1, b_g1, W_g2, b_g2, W_out, b_out):
    s1 = gamma1 / jnp.sqrt(1.0 + EPS_BN)
    w1 = W_conv1.reshape(16, 27).T * s1[None, :]
    c1 = b_conv1 * s1 + beta1
    s2 = gamma2 / jnp.sqrt(1.0 + EPS_BN)
    w2 = W_conv2.reshape(32, 144).T * s2[None, :]
    c2 = b_conv2 * s2 + beta2
    kidx = jnp.arange(288)
    wfc = W_fc[(kidx % 32) * 9 + kidx // 32]

    row = edge_index[0].astype(jnp.int32)
    col = edge_index[1].astype(jnp.int32)
    col16 = col.reshape(NW, EPT // 16, 16)
    row2d = row.reshape(E // K, K)
    col2d = col.reshape(E // K, K)
    zrows = jnp.zeros((NPAD // NS, 128), jnp.float32)

    degs = _degree(col16)
    degsT = degs.reshape(NW, N // 400, 400).transpose(1, 0, 2)

    p1 = x.reshape(N, 3, 9, 3, 9, 3).transpose(0, 2, 4, 1, 3, 5)
    p1 = p1.reshape(N * 81, 27)
    h1 = _mm_act(p1, w1, c1, "leaky", block_rows=81 * 200)
    p2 = h1.reshape(N, 3, 3, 3, 3, 16).transpose(0, 1, 3, 5, 2, 4)
    p2 = p2.reshape(N * 9, 144)
    h2 = _mm_act(p2, w2, c2, "leaky", block_rows=9 * 1000)
    h2f = h2.reshape(N, 288)

    xs1 = _fc_gcn(h2f, wfc, b_fc, W_g1, degsT)

    acc1 = _propagate(xs1, row2d, col2d, zrows)
    xs2 = _combine_next(acc1, xs1, degsT, b_g1, W_g2)

    acc2 = _propagate(xs2, row2d, col2d, zrows)
    return _head(acc2, xs2, degsT, b_g2, W_out, b_out)


# speedup vs baseline: 23.6806x; 23.6806x over previous
import functools

import jax
import jax.numpy as jnp
from jax import lax
from jax.experimental import pallas as pl
from jax.experimental.pallas import tpu as pltpu
from jax.experimental.pallas import tpu_sc as plsc

EPS_BN = 1e-5
SLOPE = 0.01

N = 10000
E = 320000
NC = 2
NS = 16
NW = NC * NS
EPT = E // NW
K = 125
NCHUNK = EPT // K
NPAD = 10240
DB = 400
ND = N // DB
DP = 512


def _leaky(v):
    return jnp.where(v >= 0, v, SLOPE * v)


def _conv1_kernel(a_ref, w_ref, b_ref, o_ref):
    b = b_ref[...][None, :]
    ys = []
    for i in range(9):
        acc = None
        for c in range(3):
            lo = c * 729 + i * 81
            t = jnp.dot(a_ref[:, lo:lo + 81], w_ref[c],
                        preferred_element_type=jnp.float32)
            acc = t if acc is None else acc + t
        ys.append(_leaky(acc + b))
    o_ref[...] = jnp.concatenate(ys, axis=1)


def _conv1(x2d, w1s, b1, block_rows=1000):
    grid = N // block_rows
    return pl.pallas_call(
        _conv1_kernel,
        grid=(grid,),
        in_specs=[
            pl.BlockSpec((block_rows, 2187), lambda i: (i, 0)),
            pl.BlockSpec((3, 81, 144), lambda i: (0, 0, 0)),
            pl.BlockSpec((144,), lambda i: (0,)),
        ],
        out_specs=pl.BlockSpec((block_rows, 1296), lambda i: (i, 0)),
        out_shape=jax.ShapeDtypeStruct((N, 1296), jnp.float32),
    )(x2d, w1s, b1)


def _conv2fc_kernel(h_ref, w2_ref, b2_ref, wfc_ref, bfc_ref, wg_ref, deg_ref,
                    o_ref):
    h2 = jnp.dot(h_ref[...], w2_ref[...], preferred_element_type=jnp.float32)
    h2 = _leaky(h2 + b2_ref[...][None, :])
    h0 = jnp.dot(h2, wfc_ref[...], preferred_element_type=jnp.float32)
    h0 = jnp.maximum(h0 + bfc_ref[...][None, :], 0.0)
    deg = jnp.sum(deg_ref[...][0], axis=0)[:DB] + 1.0
    dinv = lax.rsqrt(deg)
    xs = dinv[:, None] * jnp.dot(h0, wg_ref[...],
                                 preferred_element_type=jnp.float32)
    o_ref[...] = xs


def _conv2fc(h1, w2big, b2, wfc, bfc, wg, degs):
    return pl.pallas_call(
        _conv2fc_kernel,
        grid=(ND,),
        in_specs=[
            pl.BlockSpec((DB, 1296), lambda i: (i, 0)),
            pl.BlockSpec((1296, 288), lambda i: (0, 0)),
            pl.BlockSpec((288,), lambda i: (0,)),
            pl.BlockSpec((288, 128), lambda i: (0, 0)),
            pl.BlockSpec((128,), lambda i: (0,)),
            pl.BlockSpec((128, 128), lambda i: (0, 0)),
            pl.BlockSpec((1, NW, DP), lambda i: (i, 0, 0)),
        ],
        out_specs=pl.BlockSpec((DB, 128), lambda i: (i, 0)),
        out_shape=jax.ShapeDtypeStruct((N, 128), jnp.float32),
    )(h1, w2big, b2, wfc, bfc, wg, degs)


def _combine_kernel(acc_ref, xs_ref, deg_ref, bg_ref, wg_ref, o_ref):
    deg = jnp.sum(deg_ref[...][0], axis=0)[:DB] + 1.0
    dinv = lax.rsqrt(deg)
    tot = acc_ref[0] + acc_ref[1] + xs_ref[...]
    h = jnp.maximum(dinv[:, None] * tot + bg_ref[...][None, :], 0.0)
    xs2 = dinv[:, None] * jnp.dot(h, wg_ref[...],
                                  preferred_element_type=jnp.float32)
    o_ref[...] = xs2


def _combine_next(acc, xs, degs, bg, wg):
    return pl.pallas_call(
        _combine_kernel,
        grid=(ND,),
        in_specs=[
            pl.BlockSpec((2, DB, 128), lambda i: (0, i, 0)),
            pl.BlockSpec((DB, 128), lambda i: (i, 0)),
            pl.BlockSpec((1, NW, DP), lambda i: (i, 0, 0)),
            pl.BlockSpec((128,), lambda i: (0,)),
            pl.BlockSpec((128, 128), lambda i: (0, 0)),
        ],
        out_specs=pl.BlockSpec((DB, 128), lambda i: (i, 0)),
        out_shape=jax.ShapeDtypeStruct((N, 128), jnp.float32),
    )(acc, xs, degs, bg, wg)


def _head_kernel(acc_ref, xs_ref, deg_ref, bg_ref, wo_ref, bo_ref, o_ref):
    deg = jnp.sum(deg_ref[...][0], axis=0)[:DB] + 1.0
    dinv = lax.rsqrt(deg)
    tot = acc_ref[0] + acc_ref[1] + xs_ref[...]
    h = jnp.maximum(dinv[:, None] * tot + bg_ref[...][None, :], 0.0)
    z = jnp.dot(h, wo_ref[...], preferred_element_type=jnp.float32)
    z = z + bo_ref[...][None, :]
    z = z - jnp.max(z, axis=1, keepdims=True)
    ez = jnp.exp(z)
    o_ref[...] = ez / jnp.sum(ez, axis=1, keepdims=True)


def _head(acc, xs, degs, bg, wo, bo):
    return pl.pallas_call(
        _head_kernel,
        grid=(ND,),
        in_specs=[
            pl.BlockSpec((2, DB, 128), lambda i: (0, i, 0)),
            pl.BlockSpec((DB, 128), lambda i: (i, 0)),
            pl.BlockSpec((1, NW, DP), lambda i: (i, 0, 0)),
            pl.BlockSpec((128,), lambda i: (0,)),
            pl.BlockSpec((128, 64), lambda i: (0, 0)),
            pl.BlockSpec((64,), lambda i: (0,)),
        ],
        out_specs=pl.BlockSpec((DB, 64), lambda i: (i, 0)),
        out_shape=jax.ShapeDtypeStruct((N, 64), jnp.float32),
    )(acc, xs, degs, bg, wo, bo)


def _sc_mesh():
    return plsc.VectorSubcoreMesh(core_axis_name="c", subcore_axis_name="s",
                                  num_cores=NC, num_subcores=NS)


def _degree(col16):
    rows_per_tile = EPT // 16

    @functools.partial(
        pl.kernel,
        out_type=jax.ShapeDtypeStruct((ND, NW, DP), jnp.float32),
        mesh=_sc_mesh(),
        compiler_params=pltpu.CompilerParams(needs_layout_passes=False),
        scratch_types=[
            pltpu.VMEM((rows_per_tile, 16), jnp.int32),
            pltpu.VMEM((ND * DP,), jnp.float32),
        ],
    )
    def run(col_hbm, out_hbm, col_v, deg_v):
        c = lax.axis_index("c")
        s = lax.axis_index("s")
        wid = c * NS + s
        pltpu.sync_copy(col_hbm.at[wid], col_v)
        zeros = jnp.zeros((16,), jnp.float32)

        def zbody(i, _):
            deg_v[pl.ds(i * 16, 16)] = zeros
            return 0

        lax.fori_loop(0, ND * DP // 16, zbody, 0)
        ones = jnp.ones((16,), jnp.float32)

        def body(i, _):
            idx = col_v[i, :]
            plsc.addupdate_scatter(deg_v, [idx], ones)
            return 0

        lax.fori_loop(0, rows_per_tile, body, 0)

        for j in range(ND):
            pltpu.sync_copy(deg_v.at[pl.ds(j * DP, DP)],
                            out_hbm.at[j].at[wid])

    return run(col16)


def _propagate(xs, row2d, col2d, zrows):
    nslice = NPAD // NS
    G = 16
    NGROUP = NCHUNK // G

    @functools.partial(
        pl.kernel,
        out_type=jax.ShapeDtypeStruct((NC, NPAD, 128), jnp.float32),
        mesh=_sc_mesh(),
        compiler_params=pltpu.CompilerParams(needs_layout_passes=False),
        scratch_types=[
            pltpu.VMEM((G, K), jnp.int32),
            pltpu.VMEM((G, K), jnp.int32),
            pltpu.VMEM((K, 128), jnp.float32),
            pltpu.VMEM((K, 128), jnp.float32),
            pltpu.VMEM_SHARED((NPAD, 128), jnp.float32),
            pltpu.SemaphoreType.DMA,
            pltpu.SemaphoreType.DMA,
        ],
    )
    def run(xs_hbm, row_hbm, col_hbm, z_hbm, out_hbm,
            row_v, col_v, bufa, bufb, acc, sema, semb):
        c = lax.axis_index("c")
        s = lax.axis_index("s")
        wid = c * NS + s
        pltpu.sync_copy(z_hbm, acc.at[pl.ds(s * nslice, nslice)])
        plsc.subcore_barrier()

        def wait(buf, sem):
            pltpu.make_async_copy(xs_hbm.at[row_v.at[0]], buf, sem).wait()

        def group(g, _):
            base = wid * NCHUNK + g * G
            pltpu.sync_copy(row_hbm.at[pl.ds(base, G)], row_v)
            pltpu.sync_copy(col_hbm.at[pl.ds(base, G)], col_v)
            pltpu.async_copy(xs_hbm.at[row_v.at[0]], bufa, sema)

            def body(t, _):
                j = t * 2
                pltpu.async_copy(xs_hbm.at[row_v.at[j + 1]], bufb, semb)
                wait(bufa, sema)
                pltpu.sync_copy(bufa, acc.at[col_v.at[j]], add=True)

                @pl.when(t + 1 < G // 2)
                def _():
                    pltpu.async_copy(xs_hbm.at[row_v.at[j + 2]], bufa, sema)

                wait(bufb, semb)
                pltpu.sync_copy(bufb, acc.at[col_v.at[j + 1]], add=True)
                return 0

            lax.fori_loop(0, G // 2, body, 0)
            return 0

        lax.fori_loop(0, NGROUP, group, 0)
        plsc.subcore_barrier()
        pltpu.sync_copy(acc.at[pl.ds(s * nslice, nslice)],
                        out_hbm.at[c].at[pl.ds(s * nslice, nslice)])

    return run(xs, row2d, col2d, zrows)


def kernel(x, edge_index, W_conv1, b_conv1, gamma1, beta1,
           W_conv2, b_conv2, gamma2, beta2,
           W_fc, b_fc, W_g1, b_g1, W_g2, b_g2, W_out, b_out):
    s1 = gamma1 / jnp.sqrt(1.0 + EPS_BN)
    c1 = b_conv1 * s1 + beta1
    d27 = jnp.eye(27, dtype=jnp.float32).reshape(27, 9, 3)
    w1s = jnp.einsum('ocab,wjb->cawoj', W_conv1 * s1[:, None, None, None],
                     d27).reshape(3, 81, 144)
    b1 = jnp.repeat(c1, 9)

    s2 = gamma2 / jnp.sqrt(1.0 + EPS_BN)
    c2 = b_conv2 * s2 + beta2
    i3 = jnp.eye(3, dtype=jnp.float32)
    w2big = jnp.einsum('ucab,xp,yq->xacybupq',
                       W_conv2 * s2[:, None, None, None], i3,
                       i3).reshape(1296, 288)
    b2 = jnp.repeat(c2, 9)

    row = edge_index[0].astype(jnp.int32)
    col = edge_index[1].astype(jnp.int32)
    col_deg = col + (col // DB) * (DP - DB)
    col16 = col_deg.reshape(NW, EPT // 16, 16)
    row2d = row.reshape(E // K, K)
    col2d = col.reshape(E // K, K)
    zrows = jnp.zeros((NPAD // NS, 128), jnp.float32)

    degs = _degree(col16)

    x2d = x.reshape(N, 3 * 27 * 27)
    h1 = _conv1(x2d, w1s, b1)
    xs1 = _conv2fc(h1, w2big, b2, W_fc, b_fc, W_g1, degs)

    acc1 = _propagate(xs1, row2d, col2d, zrows)
    xs2 = _combine_next(acc1, xs1, degs, b_g1, W_g2)

    acc2 = _propagate(xs2, row2d, col2d, zrows)
    return _head(acc2, xs2, degs, b_g2, W_out, b_out)
